# baseline (speedup 1.0000x reference)
import functools

import jax
import jax.numpy as jnp
from jax import lax
from jax.experimental import pallas as pl
from jax.experimental.pallas import tpu as pltpu
from jax.experimental.pallas import tpu_sc as plsc

N = 10000
E = 320000
F = 128

NC = 2
NS = 16
NW = NC * NS
EPW = E // NW
CH = 80
NCHUNK = EPW // CH
CTILES = 10
RPC = N // CTILES

BN = 2000
GRID = N // BN


def _spmm_body(seq_hbm, src_hbm, dst_hbm, ew_hbm, out_hbm,
               srcs_v, ewa_v, ewb_v, dsta_v, dstb_v, bufa_v, bufb_v, agg_sh,
               gsa, gsb, dsa, dsb, esa, esb, ssa, ssb):
    c = lax.axis_index("c")
    s = lax.axis_index("s")
    wid = s * NC + c

    pltpu.sync_copy(src_hbm.at[wid], srcs_v)

    def zrow(i, carry):
        for j in range(8):
            bufa_v[i, pl.ds(j * 16, 16)] = jnp.zeros((16,), jnp.float32)
        return carry
    lax.fori_loop(0, CH, zrow, 0)

    @pl.when(s < CTILES)
    def _():
        for k in range(12):
            pltpu.async_copy(bufa_v, agg_sh.at[pl.ds(s * RPC + k * 80, 80)],
                             gsa)
        pltpu.async_copy(bufa_v.at[pl.ds(0, 40)],
                         agg_sh.at[pl.ds(s * RPC + 960, 40)], gsa)
        for k in range(12):
            pltpu.make_async_copy(
                bufa_v, agg_sh.at[pl.ds(s * RPC + k * 80, 80)], gsa).wait()
        pltpu.make_async_copy(
            bufa_v.at[pl.ds(0, 40)],
            agg_sh.at[pl.ds(s * RPC + 960, 40)], gsa).wait()
    plsc.subcore_barrier()

    def gather(ci, buf, sem):
        pltpu.async_copy(seq_hbm.at[srcs_v.at[ci]], buf, sem)

    def scale(ebuf, buf):
        def grp(g, inner):
            wg = ebuf[pl.ds(g * 16, 16)]
            for k in range(16):
                wvec = jnp.full((16,), wg[k], jnp.float32)
                e = g * 16 + k
                for j in range(8):
                    sl = pl.ds(j * 16, 16)
                    buf[e, sl] = buf[e, sl] * wvec
            return inner
        lax.fori_loop(0, CH // 16, grp, 0)

    def gwait(buf, sem):
        pltpu.make_async_copy(seq_hbm.at[srcs_v.at[0]], buf, sem).wait()

    def dload(ci, dbuf, sem):
        pltpu.async_copy(dst_hbm.at[wid, ci], dbuf, sem)

    def dwait(dbuf, sem):
        pltpu.make_async_copy(dst_hbm.at[wid, 0], dbuf, sem).wait()

    def eload(ci, ebuf, sem):
        pltpu.async_copy(ew_hbm.at[wid, ci], ebuf, sem)

    def ewait(ebuf, sem):
        pltpu.make_async_copy(ew_hbm.at[wid, 0], ebuf, sem).wait()

    def scatter(dbuf, buf, sem):
        pltpu.async_copy(buf, agg_sh.at[dbuf], sem, add=True)

    def swait(dbuf, buf, sem):
        pltpu.make_async_copy(buf, agg_sh.at[dbuf], sem).wait()

    pltpu.sync_copy(dst_hbm.at[wid, 0], dsta_v)
    pltpu.sync_copy(dst_hbm.at[wid, 1], dstb_v)
    pltpu.sync_copy(ew_hbm.at[wid, 0], ewa_v)
    pltpu.sync_copy(ew_hbm.at[wid, 1], ewb_v)
    gather(0, bufa_v, gsa)

    def pair(p, carry):
        c0 = 2 * p
        c1 = c0 + 1

        @pl.when(p > 0)
        def _():
            swait(dstb_v, bufb_v, ssb)
        gather(c1, bufb_v, gsb)
        gwait(bufa_v, gsa)

        @pl.when(p > 0)
        def _():
            ewait(ewa_v, esa)
        scale(ewa_v, bufa_v)
        eload(c0 + 2, ewa_v, esa)

        @pl.when(p > 0)
        def _():
            dwait(dsta_v, dsa)
        scatter(dsta_v, bufa_v, ssa)
        gather_dst = dload(c0 + 2, dsta_v, dsa)
        gwait(bufb_v, gsb)

        @pl.when(p > 0)
        def _():
            ewait(ewb_v, esb)
        scale(ewb_v, bufb_v)

        @pl.when(p > 0)
        def _():
            dwait(dstb_v, dsb)

        @pl.when(p < (NCHUNK - 3) // 2)
        def _():
            eload(c1 + 2, ewb_v, esb)
            dload(c1 + 2, dstb_v, dsb)
        swait(dsta_v, bufa_v, ssa)
        gather(c0 + 2, bufa_v, gsa)
        scatter(dstb_v, bufb_v, ssb)
        return carry
    lax.fori_loop(0, (NCHUNK - 1) // 2, pair, 0)

    swait(dstb_v, bufb_v, ssb)
    gwait(bufa_v, gsa)
    ewait(ewa_v, esa)
    scale(ewa_v, bufa_v)
    dwait(dsta_v, dsa)
    pltpu.sync_copy(bufa_v, agg_sh.at[dsta_v], add=True)

    plsc.subcore_barrier()

    @pl.when(s < CTILES)
    def _():
        bufs = [bufa_v, bufb_v]
        sin = [gsa, gsb]
        sout = [ssa, ssb]
        for k in range(13):
            n = 80 if k < 12 else 40
            r0 = s * RPC + k * 80
            b = bufs[k % 2].at[pl.ds(0, n)]
            if k >= 2:
                np_ = 80 if k - 2 < 12 else 40
                rp = s * RPC + (k - 2) * 80
                bp = bufs[k % 2].at[pl.ds(0, np_)]
                pltpu.make_async_copy(
                    bp, out_hbm.at[c, pl.ds(rp, np_)], sout[k % 2]).wait()
            pltpu.make_async_copy(
                agg_sh.at[pl.ds(r0, n)], b, sin[k % 2]).start()
            pltpu.make_async_copy(
                agg_sh.at[pl.ds(r0, n)], b, sin[k % 2]).wait()
            pltpu.make_async_copy(
                b, out_hbm.at[c, pl.ds(r0, n)], sout[k % 2]).start()
        for k in (11, 12):
            n = 80 if k < 12 else 40
            r0 = s * RPC + k * 80
            b = bufs[k % 2].at[pl.ds(0, n)]
            pltpu.make_async_copy(
                b, out_hbm.at[c, pl.ds(r0, n)], sout[k % 2]).wait()


_spmm = pl.kernel(
    _spmm_body,
    out_type=jax.ShapeDtypeStruct((NC, N, F), jnp.float32),
    mesh=plsc.VectorSubcoreMesh(core_axis_name="c", subcore_axis_name="s"),
    scratch_types=[
        pltpu.VMEM((NCHUNK, CH), jnp.int32),
        pltpu.VMEM((CH,), jnp.float32),
        pltpu.VMEM((CH,), jnp.float32),
        pltpu.VMEM((CH,), jnp.int32),
        pltpu.VMEM((CH,), jnp.int32),
        pltpu.VMEM((CH, F), jnp.float32),
        pltpu.VMEM((CH, F), jnp.float32),
        pltpu.VMEM_SHARED((N, F), jnp.float32),
        pltpu.SemaphoreType.DMA,
        pltpu.SemaphoreType.DMA,
        pltpu.SemaphoreType.DMA,
        pltpu.SemaphoreType.DMA,
        pltpu.SemaphoreType.DMA,
        pltpu.SemaphoreType.DMA,
        pltpu.SemaphoreType.DMA,
        pltpu.SemaphoreType.DMA,
    ],
)


def _k1_body(x_ref, evo_ref, w0_ref, b0_ref, cw0_ref, w1_ref, b1_ref,
             seq0_ref, glob_ref):
    local = jnp.maximum(
        jnp.dot(x_ref[...], w0_ref[...], preferred_element_type=jnp.float32)
        + b0_ref[...], 0.0)
    seq0_ref[...] = jnp.dot(local, cw0_ref[...],
                            preferred_element_type=jnp.float32)
    glob_ref[...] = jnp.maximum(
        jnp.dot(evo_ref[...], w1_ref[...], preferred_element_type=jnp.float32)
        + b1_ref[...], 0.0)


def _full(shape):
    return pl.BlockSpec(shape, lambda i: (0,) * len(shape))


_k1 = pl.pallas_call(
    _k1_body,
    grid=(GRID,),
    in_specs=[
        pl.BlockSpec((BN, F), lambda i: (i, 0)),
        pl.BlockSpec((BN, 1024), lambda i: (i, 0)),
        _full((F, F)), _full((1, F)), _full((F, F)),
        _full((1024, F)), _full((1, F)),
    ],
    out_specs=[
        pl.BlockSpec((BN, F), lambda i: (i, 0)),
        pl.BlockSpec((BN, F), lambda i: (i, 0)),
    ],
    out_shape=[
        jax.ShapeDtypeStruct((N, F), jnp.float32),
        jax.ShapeDtypeStruct((N, F), jnp.float32),
    ],
)


def _k2_body(p_ref, cw1_ref, local1_ref, seq1_ref):
    local1 = jnp.maximum(p_ref[0] + p_ref[1], 0.0)
    local1_ref[...] = local1
    seq1_ref[...] = jnp.dot(local1, cw1_ref[...],
                            preferred_element_type=jnp.float32)


_k2 = pl.pallas_call(
    _k2_body,
    grid=(GRID,),
    in_specs=[
        pl.BlockSpec((NC, BN, F), lambda i: (0, i, 0)),
        _full((F, F)),
    ],
    out_specs=[
        pl.BlockSpec((BN, F), lambda i: (i, 0)),
        pl.BlockSpec((BN, F), lambda i: (i, 0)),
    ],
    out_shape=[
        jax.ShapeDtypeStruct((N, F), jnp.float32),
        jax.ShapeDtypeStruct((N, F), jnp.float32),
    ],
)


def _k3_body(p_ref, local1_ref, glob_ref,
             fc2a_ref, fc2b_ref, fc2_b_ref,
             fc3a_ref, fc3b_ref, fc3_b_ref,
             proj_w_ref, proj_b_ref, proj1_w_ref, proj1_b_ref,
             proj2_w_ref, proj2_b_ref,
             ec1_w_ref, ec1_b_ref, ec2_w_ref, ec2_b_ref,
             ec3_w_ref, ec3_b_ref, ec4_w_ref, ec4_b_ref,
             out_ref, ec_ref, hsum_ref):
    i = pl.program_id(0)

    local2 = jnp.maximum(p_ref[0] + p_ref[1], 0.0)
    local1 = local1_ref[...]
    t = jnp.maximum(
        jnp.dot(local1, fc2a_ref[...], preferred_element_type=jnp.float32)
        + jnp.dot(local2, fc2b_ref[...], preferred_element_type=jnp.float32)
        + fc2_b_ref[...], 0.0)
    enz = jnp.maximum(
        jnp.dot(glob_ref[...], fc3a_ref[...],
                preferred_element_type=jnp.float32)
        + jnp.dot(t, fc3b_ref[...], preferred_element_type=jnp.float32)
        + fc3_b_ref[...], 0.0)
    inner = jnp.maximum(
        jnp.dot(enz, proj_w_ref[...], preferred_element_type=jnp.float32)
        + proj_b_ref[...], 0.0)
    inner = jnp.maximum(
        jnp.dot(inner, proj1_w_ref[...], preferred_element_type=jnp.float32)
        + proj1_b_ref[...], 0.0)
    out_ref[...] = (
        jnp.dot(inner, proj2_w_ref[...], preferred_element_type=jnp.float32)
        + proj2_b_ref[...])

    h = jnp.maximum(
        jnp.dot(enz, ec1_w_ref[...], preferred_element_type=jnp.float32)
        + ec1_b_ref[...], 0.0)
    h = jnp.maximum(
        jnp.dot(h, ec2_w_ref[...], preferred_element_type=jnp.float32)
        + ec2_b_ref[...], 0.0)
    h = jnp.maximum(
        jnp.dot(h, ec3_w_ref[...], preferred_element_type=jnp.float32)
        + ec3_b_ref[...], 0.0)
    part = jnp.sum(h, axis=0, keepdims=True)

    @pl.when(i == 0)
    def _():
        hsum_ref[...] = part

    @pl.when(i > 0)
    def _():
        hsum_ref[...] = hsum_ref[...] + part

    @pl.when(i == GRID - 1)
    def _():
        hmean = hsum_ref[...] * (1.0 / N)
        ec_ref[...] = jnp.maximum(
            jnp.dot(hmean, ec4_w_ref[...], preferred_element_type=jnp.float32)
            + ec4_b_ref[...], 0.0)


_k3 = pl.pallas_call(
    _k3_body,
    grid=(GRID,),
    in_specs=[
        pl.BlockSpec((NC, BN, F), lambda i: (0, i, 0)),
        pl.BlockSpec((BN, F), lambda i: (i, 0)),
        pl.BlockSpec((BN, F), lambda i: (i, 0)),
        _full((F, F)), _full((F, F)), _full((1, F)),
        _full((F, F)), _full((F, F)), _full((1, F)),
        _full((F, 64)), _full((1, 64)), _full((64, F)), _full((1, F)),
        _full((F, F)), _full((1, F)),
        _full((F, 256)), _full((1, 256)), _full((256, F)), _full((1, F)),
        _full((F, F)), _full((1, F)), _full((F, 1024)), _full((1, 1024)),
    ],
    out_specs=[
        pl.BlockSpec((BN, F), lambda i: (i, 0)),
        pl.BlockSpec((1, 1024), lambda i: (0, 0)),
    ],
    out_shape=[
        jax.ShapeDtypeStruct((N, F), jnp.float32),
        jax.ShapeDtypeStruct((1, 1024), jnp.float32),
    ],
    scratch_shapes=[pltpu.VMEM((1, F), jnp.float32)],
)


def kernel(x, edge_index, edge_weight, evo_fea, params):
    p = params
    dst = edge_index[0].reshape(NW, NCHUNK, CH)
    src = edge_index[1].reshape(NW, NCHUNK, CH)
    ew = edge_weight.reshape(NW, NCHUNK, CH)

    def row(b):
        return b.reshape(1, -1)

    seq0, glob = _k1(x, evo_fea, p['fc0_w'], row(p['fc0_b']),
                     p['conv_w'][0], p['fc1_w'], row(p['fc1_b']))
    p0 = _spmm(seq0, src, dst, ew)
    local1, seq1 = _k2(p0, p['conv_w'][1])
    p1 = _spmm(seq1, src, dst, ew)
    out, ec = _k3(
        p1, local1, glob,
        p['fc2_w'][:F], p['fc2_w'][F:], row(p['fc2_b']),
        p['fc3_w'][:F], p['fc3_w'][F:], row(p['fc3_b']),
        p['proj_w'], row(p['proj_b']), p['proj1_w'], row(p['proj1_b']),
        p['proj2_w'], row(p['proj2_b']),
        p['ec1_w'], row(p['ec1_b']), p['ec2_w'], row(p['ec2_b']),
        p['ec3_w'], row(p['ec3_b']), p['ec4_w'], row(p['ec4_b']))
    return (out, ec.reshape(1024,))


# speedup vs baseline: 1.0268x; 1.0268x over previous
import functools

import jax
import jax.numpy as jnp
from jax import lax
from jax.experimental import pallas as pl
from jax.experimental.pallas import tpu as pltpu
from jax.experimental.pallas import tpu_sc as plsc

N = 10000
E = 320000
F = 128

NC = 2
NS = 16
NW = NC * NS
EPW = E // NW
CH = 80
NCHUNK = EPW // CH
CTILES = 10
RPC = N // CTILES

BN = 2000
GRID = N // BN


def _spmm_body(seq_hbm, src_hbm, dst_hbm, ew_hbm, out_hbm,
               srcs_v, ewa_v, ewb_v, dsta_v, dstb_v, bufa_v, bufb_v, agg_sh,
               gsa, gsb, dsa, dsb, esa, esb, ssa, ssb):
    c = lax.axis_index("c")
    s = lax.axis_index("s")
    wid = s * NC + c

    pltpu.sync_copy(src_hbm.at[wid], srcs_v)

    def zrow(i, carry):
        for j in range(8):
            bufa_v[i, pl.ds(j * 16, 16)] = jnp.zeros((16,), jnp.float32)
        return carry
    lax.fori_loop(0, CH, zrow, 0)

    @pl.when(s < CTILES)
    def _():
        for k in range(12):
            pltpu.async_copy(bufa_v, agg_sh.at[pl.ds(s * RPC + k * 80, 80)],
                             gsa)
        pltpu.async_copy(bufa_v.at[pl.ds(0, 40)],
                         agg_sh.at[pl.ds(s * RPC + 960, 40)], gsa)
        for k in range(12):
            pltpu.make_async_copy(
                bufa_v, agg_sh.at[pl.ds(s * RPC + k * 80, 80)], gsa).wait()
        pltpu.make_async_copy(
            bufa_v.at[pl.ds(0, 40)],
            agg_sh.at[pl.ds(s * RPC + 960, 40)], gsa).wait()
    plsc.subcore_barrier()

    def gather(ci, buf, sem):
        pltpu.async_copy(seq_hbm.at[srcs_v.at[ci]], buf, sem)

    def scale(ebuf, buf):
        def grp(g, inner):
            wg = ebuf[pl.ds(g * 16, 16)]
            for k in range(16):
                wvec = jnp.full((16,), wg[k], jnp.float32)
                e = g * 16 + k
                for j in range(8):
                    sl = pl.ds(j * 16, 16)
                    buf[e, sl] = buf[e, sl] * wvec
            return inner
        lax.fori_loop(0, CH // 16, grp, 0)

    def gwait(buf, sem):
        pltpu.make_async_copy(seq_hbm.at[srcs_v.at[0]], buf, sem).wait()

    def dload(ci, dbuf, sem):
        pltpu.async_copy(dst_hbm.at[wid, ci], dbuf, sem)

    def dwait(dbuf, sem):
        pltpu.make_async_copy(dst_hbm.at[wid, 0], dbuf, sem).wait()

    def eload(ci, ebuf, sem):
        pltpu.async_copy(ew_hbm.at[wid, ci], ebuf, sem)

    def ewait(ebuf, sem):
        pltpu.make_async_copy(ew_hbm.at[wid, 0], ebuf, sem).wait()

    def scatter(dbuf, buf, sem):
        pltpu.async_copy(buf, agg_sh.at[dbuf], sem, add=True)

    def swait(dbuf, buf, sem):
        pltpu.make_async_copy(buf, agg_sh.at[dbuf], sem).wait()

    pltpu.sync_copy(dst_hbm.at[wid, 0], dsta_v)
    pltpu.sync_copy(dst_hbm.at[wid, 1], dstb_v)
    pltpu.sync_copy(ew_hbm.at[wid, 0], ewa_v)
    pltpu.sync_copy(ew_hbm.at[wid, 1], ewb_v)
    gather(0, bufa_v, gsa)

    def pair(p, carry):
        c0 = 2 * p
        c1 = c0 + 1

        @pl.when(p > 0)
        def _():
            swait(dstb_v, bufb_v, ssb)
        gather(c1, bufb_v, gsb)
        gwait(bufa_v, gsa)

        @pl.when(p > 0)
        def _():
            ewait(ewa_v, esa)
        scale(ewa_v, bufa_v)
        eload(c0 + 2, ewa_v, esa)

        @pl.when(p > 0)
        def _():
            dwait(dsta_v, dsa)
        scatter(dsta_v, bufa_v, ssa)
        gather_dst = dload(c0 + 2, dsta_v, dsa)
        gwait(bufb_v, gsb)

        @pl.when(p > 0)
        def _():
            ewait(ewb_v, esb)
        scale(ewb_v, bufb_v)

        @pl.when(p > 0)
        def _():
            dwait(dstb_v, dsb)

        @pl.when(p < (NCHUNK - 3) // 2)
        def _():
            eload(c1 + 2, ewb_v, esb)
            dload(c1 + 2, dstb_v, dsb)
        swait(dsta_v, bufa_v, ssa)
        gather(c0 + 2, bufa_v, gsa)
        scatter(dstb_v, bufb_v, ssb)
        return carry
    lax.fori_loop(0, (NCHUNK - 1) // 2, pair, 0)

    swait(dstb_v, bufb_v, ssb)
    gwait(bufa_v, gsa)
    ewait(ewa_v, esa)
    scale(ewa_v, bufa_v)
    dwait(dsta_v, dsa)
    pltpu.sync_copy(bufa_v, agg_sh.at[dsta_v], add=True)

    plsc.subcore_barrier()

    @pl.when(s < CTILES)
    def _():
        bufs = [bufa_v, bufb_v]
        sin = [gsa, gsb]
        sout = [ssa, ssb]
        for k in range(13):
            n = 80 if k < 12 else 40
            r0 = s * RPC + k * 80
            b = bufs[k % 2].at[pl.ds(0, n)]
            if k >= 2:
                np_ = 80 if k - 2 < 12 else 40
                rp = s * RPC + (k - 2) * 80
                bp = bufs[k % 2].at[pl.ds(0, np_)]
                pltpu.make_async_copy(
                    bp, out_hbm.at[c, pl.ds(rp, np_)], sout[k % 2]).wait()
            pltpu.make_async_copy(
                agg_sh.at[pl.ds(r0, n)], b, sin[k % 2]).start()
            pltpu.make_async_copy(
                agg_sh.at[pl.ds(r0, n)], b, sin[k % 2]).wait()
            pltpu.make_async_copy(
                b, out_hbm.at[c, pl.ds(r0, n)], sout[k % 2]).start()
        for k in (11, 12):
            n = 80 if k < 12 else 40
            r0 = s * RPC + k * 80
            b = bufs[k % 2].at[pl.ds(0, n)]
            pltpu.make_async_copy(
                b, out_hbm.at[c, pl.ds(r0, n)], sout[k % 2]).wait()


_spmm = pl.kernel(
    _spmm_body,
    out_type=jax.ShapeDtypeStruct((NC, N, F), jnp.float32),
    mesh=plsc.VectorSubcoreMesh(core_axis_name="c", subcore_axis_name="s"),
    scratch_types=[
        pltpu.VMEM((NCHUNK, CH), jnp.int32),
        pltpu.VMEM((CH,), jnp.float32),
        pltpu.VMEM((CH,), jnp.float32),
        pltpu.VMEM((CH,), jnp.int32),
        pltpu.VMEM((CH,), jnp.int32),
        pltpu.VMEM((CH, F), jnp.float32),
        pltpu.VMEM((CH, F), jnp.float32),
        pltpu.VMEM_SHARED((N, F), jnp.float32),
        pltpu.SemaphoreType.DMA,
        pltpu.SemaphoreType.DMA,
        pltpu.SemaphoreType.DMA,
        pltpu.SemaphoreType.DMA,
        pltpu.SemaphoreType.DMA,
        pltpu.SemaphoreType.DMA,
        pltpu.SemaphoreType.DMA,
        pltpu.SemaphoreType.DMA,
    ],
)


def _k1_body(x_ref, w0_ref, b0_ref, cw0_ref, seq0_ref):
    local = jnp.maximum(
        jnp.dot(x_ref[...], w0_ref[...], preferred_element_type=jnp.float32)
        + b0_ref[...], 0.0)
    seq0_ref[...] = jnp.dot(local, cw0_ref[...],
                            preferred_element_type=jnp.float32)


def _kglob_body(evo_ref, w1_ref, b1_ref, glob_ref):
    glob_ref[...] = jnp.maximum(
        jnp.dot(evo_ref[...], w1_ref[...], preferred_element_type=jnp.float32)
        + b1_ref[...], 0.0)


def _full(shape):
    return pl.BlockSpec(shape, lambda i: (0,) * len(shape))


_k1 = pl.pallas_call(
    _k1_body,
    grid=(GRID,),
    in_specs=[
        pl.BlockSpec((BN, F), lambda i: (i, 0)),
        _full((F, F)), _full((1, F)), _full((F, F)),
    ],
    out_specs=pl.BlockSpec((BN, F), lambda i: (i, 0)),
    out_shape=jax.ShapeDtypeStruct((N, F), jnp.float32),
)

_kglob = pl.pallas_call(
    _kglob_body,
    grid=(GRID,),
    in_specs=[
        pl.BlockSpec((BN, 1024), lambda i: (i, 0)),
        _full((1024, F)), _full((1, F)),
    ],
    out_specs=pl.BlockSpec((BN, F), lambda i: (i, 0)),
    out_shape=jax.ShapeDtypeStruct((N, F), jnp.float32),
)


def _k2_body(p_ref, cw1_ref, local1_ref, seq1_ref):
    local1 = jnp.maximum(p_ref[0] + p_ref[1], 0.0)
    local1_ref[...] = local1
    seq1_ref[...] = jnp.dot(local1, cw1_ref[...],
                            preferred_element_type=jnp.float32)


_k2 = pl.pallas_call(
    _k2_body,
    grid=(GRID,),
    in_specs=[
        pl.BlockSpec((NC, BN, F), lambda i: (0, i, 0)),
        _full((F, F)),
    ],
    out_specs=[
        pl.BlockSpec((BN, F), lambda i: (i, 0)),
        pl.BlockSpec((BN, F), lambda i: (i, 0)),
    ],
    out_shape=[
        jax.ShapeDtypeStruct((N, F), jnp.float32),
        jax.ShapeDtypeStruct((N, F), jnp.float32),
    ],
)


def _k3_body(p_ref, local1_ref, glob_ref,
             fc2a_ref, fc2b_ref, fc2_b_ref,
             fc3a_ref, fc3b_ref, fc3_b_ref,
             proj_w_ref, proj_b_ref, proj1_w_ref, proj1_b_ref,
             proj2_w_ref, proj2_b_ref,
             ec1_w_ref, ec1_b_ref, ec2_w_ref, ec2_b_ref,
             ec3_w_ref, ec3_b_ref, ec4_w_ref, ec4_b_ref,
             out_ref, ec_ref, hsum_ref):
    i = pl.program_id(0)

    local2 = jnp.maximum(p_ref[0] + p_ref[1], 0.0)
    local1 = local1_ref[...]
    t = jnp.maximum(
        jnp.dot(local1, fc2a_ref[...], preferred_element_type=jnp.float32)
        + jnp.dot(local2, fc2b_ref[...], preferred_element_type=jnp.float32)
        + fc2_b_ref[...], 0.0)
    enz = jnp.maximum(
        jnp.dot(glob_ref[...], fc3a_ref[...],
                preferred_element_type=jnp.float32)
        + jnp.dot(t, fc3b_ref[...], preferred_element_type=jnp.float32)
        + fc3_b_ref[...], 0.0)
    inner = jnp.maximum(
        jnp.dot(enz, proj_w_ref[...], preferred_element_type=jnp.float32)
        + proj_b_ref[...], 0.0)
    inner = jnp.maximum(
        jnp.dot(inner, proj1_w_ref[...], preferred_element_type=jnp.float32)
        + proj1_b_ref[...], 0.0)
    out_ref[...] = (
        jnp.dot(inner, proj2_w_ref[...], preferred_element_type=jnp.float32)
        + proj2_b_ref[...])

    h = jnp.maximum(
        jnp.dot(enz, ec1_w_ref[...], preferred_element_type=jnp.float32)
        + ec1_b_ref[...], 0.0)
    h = jnp.maximum(
        jnp.dot(h, ec2_w_ref[...], preferred_element_type=jnp.float32)
        + ec2_b_ref[...], 0.0)
    h = jnp.maximum(
        jnp.dot(h, ec3_w_ref[...], preferred_element_type=jnp.float32)
        + ec3_b_ref[...], 0.0)
    part = jnp.sum(h, axis=0, keepdims=True)

    @pl.when(i == 0)
    def _():
        hsum_ref[...] = part

    @pl.when(i > 0)
    def _():
        hsum_ref[...] = hsum_ref[...] + part

    @pl.when(i == GRID - 1)
    def _():
        hmean = hsum_ref[...] * (1.0 / N)
        ec_ref[...] = jnp.maximum(
            jnp.dot(hmean, ec4_w_ref[...], preferred_element_type=jnp.float32)
            + ec4_b_ref[...], 0.0)


_k3 = pl.pallas_call(
    _k3_body,
    grid=(GRID,),
    in_specs=[
        pl.BlockSpec((NC, BN, F), lambda i: (0, i, 0)),
        pl.BlockSpec((BN, F), lambda i: (i, 0)),
        pl.BlockSpec((BN, F), lambda i: (i, 0)),
        _full((F, F)), _full((F, F)), _full((1, F)),
        _full((F, F)), _full((F, F)), _full((1, F)),
        _full((F, 64)), _full((1, 64)), _full((64, F)), _full((1, F)),
        _full((F, F)), _full((1, F)),
        _full((F, 256)), _full((1, 256)), _full((256, F)), _full((1, F)),
        _full((F, F)), _full((1, F)), _full((F, 1024)), _full((1, 1024)),
    ],
    out_specs=[
        pl.BlockSpec((BN, F), lambda i: (i, 0)),
        pl.BlockSpec((1, 1024), lambda i: (0, 0)),
    ],
    out_shape=[
        jax.ShapeDtypeStruct((N, F), jnp.float32),
        jax.ShapeDtypeStruct((1, 1024), jnp.float32),
    ],
    scratch_shapes=[pltpu.VMEM((1, F), jnp.float32)],
)


def kernel(x, edge_index, edge_weight, evo_fea, params):
    p = params
    dst = edge_index[0].reshape(NW, NCHUNK, CH)
    src = edge_index[1].reshape(NW, NCHUNK, CH)
    ew = edge_weight.reshape(NW, NCHUNK, CH)

    def row(b):
        return b.reshape(1, -1)

    seq0 = _k1(x, p['fc0_w'], row(p['fc0_b']), p['conv_w'][0])
    p0 = _spmm(seq0, src, dst, ew)
    glob = _kglob(evo_fea, p['fc1_w'], row(p['fc1_b']))
    local1, seq1 = _k2(p0, p['conv_w'][1])
    p1 = _spmm(seq1, src, dst, ew)
    out, ec = _k3(
        p1, local1, glob,
        p['fc2_w'][:F], p['fc2_w'][F:], row(p['fc2_b']),
        p['fc3_w'][:F], p['fc3_w'][F:], row(p['fc3_b']),
        p['proj_w'], row(p['proj_b']), p['proj1_w'], row(p['proj1_b']),
        p['proj2_w'], row(p['proj2_b']),
        p['ec1_w'], row(p['ec1_b']), p['ec2_w'], row(p['ec2_b']),
        p['ec3_w'], row(p['ec3_b']), p['ec4_w'], row(p['ec4_b']))
    return (out, ec.reshape(1024,))
